# baseline (speedup 1.0000x reference)
import jax
import jax.numpy as jnp
from jax import lax
from jax.experimental import pallas as pl
from jax.experimental.pallas import tpu as pltpu
from jax.experimental.pallas import tpu_sc as plsc

_B = 4096
_NC = 2
_NS = 16
_NW = _NC * _NS
_BPW = _B // _NW
_L = 16


def _body(idx_hbm, betas_hbm, go_hbm, bp_hbm, tr_hbm,
          out_b, out_go, out_bp, out_tr,
          idx_v, zidx_v, b_v, go_v, bp_v, tr_v,
          sem_b, sem_go, sem_bp, sem_tr):
    wid = lax.axis_index("s") * _NC + lax.axis_index("c")
    base = wid * _BPW
    pltpu.sync_copy(idx_hbm.at[pl.ds(base, _BPW)], idx_v)
    for i in range(_BPW // _L):
        zidx_v[pl.ds(i * _L, _L)] = jnp.zeros((_L,), jnp.int32)
    cp_bp = pltpu.async_copy(bp_hbm.at[idx_v], bp_v, sem_bp)
    cp_go = pltpu.async_copy(go_hbm.at[idx_v], go_v, sem_go)
    cp_tr = pltpu.async_copy(tr_hbm.at[idx_v], tr_v, sem_tr)
    cp_b = pltpu.async_copy(betas_hbm.at[zidx_v], b_v, sem_b)
    cp_go.wait()
    pltpu.sync_copy(go_v, out_go.at[pl.ds(base, _BPW)])
    cp_tr.wait()
    pltpu.sync_copy(tr_v, out_tr.at[pl.ds(base, _BPW)])
    cp_b.wait()
    pltpu.sync_copy(b_v, out_b.at[pl.ds(base, _BPW)])
    cp_bp.wait()
    pltpu.sync_copy(bp_v, out_bp.at[pl.ds(base, _BPW)])


def kernel(idx, betas, global_orient, body_pose, transl):
    mesh = plsc.VectorSubcoreMesh(core_axis_name="c", subcore_axis_name="s")
    k = pl.kernel(
        _body,
        mesh=mesh,
        compiler_params=pltpu.CompilerParams(use_tc_tiling_on_sc=False),
        out_type=(
            jax.ShapeDtypeStruct((_B, 10), jnp.float32),
            jax.ShapeDtypeStruct((_B, 3), jnp.float32),
            jax.ShapeDtypeStruct((_B, 69), jnp.float32),
            jax.ShapeDtypeStruct((_B, 3), jnp.float32),
        ),
        scratch_types=[
            pltpu.VMEM((_BPW,), jnp.int32),
            pltpu.VMEM((_BPW,), jnp.int32),
            pltpu.VMEM((_BPW, 10), jnp.float32),
            pltpu.VMEM((_BPW, 3), jnp.float32),
            pltpu.VMEM((_BPW, 69), jnp.float32),
            pltpu.VMEM((_BPW, 3), jnp.float32),
            pltpu.SemaphoreType.DMA,
            pltpu.SemaphoreType.DMA,
            pltpu.SemaphoreType.DMA,
            pltpu.SemaphoreType.DMA,
        ],
    )
    return k(idx.astype(jnp.int32), betas, global_orient, body_pose, transl)


# speedup vs baseline: 2.7004x; 2.7004x over previous
import jax
import jax.numpy as jnp
from jax import lax
from jax.experimental import pallas as pl
from jax.experimental.pallas import tpu as pltpu
from jax.experimental.pallas import tpu_sc as plsc

_B = 4096
_NC = 2
_NS = 16
_NW = _NC * _NS
_BPW = _B // _NW
_L = 16


def _body(idx_hbm, betas_hbm, go_hbm, bp_hbm, tr_hbm,
          out_b, out_go, out_bp, out_tr,
          idx_v, b_row, go_v, bp_v, tr_v, sem):
    wid = lax.axis_index("s") * _NC + lax.axis_index("c")
    base = wid * _BPW

    pltpu.sync_copy(idx_hbm.at[pl.ds(base, _BPW)], idx_v)

    pltpu.sync_copy(betas_hbm, b_row)
    pending = []
    for j in range(_BPW):
        pending.append(
            pltpu.async_copy(b_row, out_b.at[pl.ds(base + j, 1)], sem))

    for g in range(_BPW // _L):
        v = idx_v[pl.ds(g * _L, _L)]
        for l in range(_L):
            r = g * _L + l
            s = v[l]
            pending.append(
                pltpu.async_copy(go_hbm.at[pl.ds(s, 1)], go_v.at[pl.ds(r, 1)], sem))
            pending.append(
                pltpu.async_copy(bp_hbm.at[pl.ds(s, 1)], bp_v.at[pl.ds(r, 1)], sem))
            pending.append(
                pltpu.async_copy(tr_hbm.at[pl.ds(s, 1)], tr_v.at[pl.ds(r, 1)], sem))
    for h in pending:
        h.wait()

    pltpu.sync_copy(go_v, out_go.at[pl.ds(base, _BPW)])
    pltpu.sync_copy(bp_v, out_bp.at[pl.ds(base, _BPW)])
    pltpu.sync_copy(tr_v, out_tr.at[pl.ds(base, _BPW)])


def kernel(idx, betas, global_orient, body_pose, transl):
    mesh = plsc.VectorSubcoreMesh(core_axis_name="c", subcore_axis_name="s")
    k = pl.kernel(
        _body,
        mesh=mesh,
        out_type=(
            jax.ShapeDtypeStruct((_B, 10), jnp.float32),
            jax.ShapeDtypeStruct((_B, 3), jnp.float32),
            jax.ShapeDtypeStruct((_B, 69), jnp.float32),
            jax.ShapeDtypeStruct((_B, 3), jnp.float32),
        ),
        scratch_types=[
            pltpu.VMEM((_BPW,), jnp.int32),
            pltpu.VMEM((1, 10), jnp.float32),
            pltpu.VMEM((_BPW, 3), jnp.float32),
            pltpu.VMEM((_BPW, 69), jnp.float32),
            pltpu.VMEM((_BPW, 3), jnp.float32),
            pltpu.SemaphoreType.DMA,
        ],
    )
    return k(idx.astype(jnp.int32), betas, global_orient, body_pose, transl)
